# Initial kernel scaffold; baseline (speedup 1.0000x reference)
import jax
import jax.numpy as jnp
from jax.experimental import pallas as pl


def kernel(node_feats, edge_feats, edge_index, Wn, bn, gn, betan, We, be, ge, betae, convW, convb, Wd1, bd1, Wd2, bd2):
    raise NotImplementedError("write your pallas kernel here")


# speedup vs baseline: 8.2964x; 8.2964x over previous
import functools

import jax
import jax.numpy as jnp
from jax import lax
from jax.experimental import pallas as pl
from jax.experimental.pallas import tpu as pltpu
from jax.experimental.pallas import tpu_sc as plsc

_N = 10000
_E = 320000
_DF = 128
_H = 64
_OUT = 3
_P = 4

_NP = 10240
_NC = 2
_NS = 16
_NW = _NC * _NS
_RPS = _NP // _NS

_EW = _E // _NW
_CH = 80
_NCHUNK = _EW // _CH

_DEGW = 8

_mesh = plsc.VectorSubcoreMesh(core_axis_name="c", subcore_axis_name="s")
_sc_params = pltpu.CompilerParams(use_tc_tiling_on_sc=False)


def _deg_body(src3, dst3, ones_hbm, zdeg, degp, sdego, sdegi, src_v, dst_v,
              ones_v):
    cid = lax.axis_index("c")
    sid = lax.axis_index("s")
    wid = sid * _NC + cid
    zslab = pl.ds(sid * _RPS, _RPS)
    pltpu.sync_copy(zdeg.at[zslab], sdego.at[zslab])
    pltpu.sync_copy(zdeg.at[zslab], sdegi.at[zslab])
    pltpu.sync_copy(src3.at[wid], src_v)
    pltpu.sync_copy(dst3.at[wid], dst_v)
    pltpu.sync_copy(ones_hbm, ones_v)
    plsc.subcore_barrier()

    def body(i, carry):
        pltpu.sync_copy(ones_v, sdego.at[src_v.at[i]], add=True)
        pltpu.sync_copy(ones_v, sdegi.at[dst_v.at[i]], add=True)
        return carry

    lax.fori_loop(0, _NCHUNK, body, 0)
    plsc.subcore_barrier()
    pltpu.sync_copy(sdego.at[zslab], degp.at[cid, 0, zslab])
    pltpu.sync_copy(sdegi.at[zslab], degp.at[cid, 1, zslab])


_deg_call = pl.kernel(
    _deg_body,
    out_type=jax.ShapeDtypeStruct((_NC, 2, _NP, _DEGW), jnp.float32),
    mesh=_mesh,
    scratch_types=[
        pltpu.VMEM_SHARED((_NP, _DEGW), jnp.float32),
        pltpu.VMEM_SHARED((_NP, _DEGW), jnp.float32),
        pltpu.VMEM((_NCHUNK, _CH), jnp.int32),
        pltpu.VMEM((_NCHUNK, _CH), jnp.int32),
        pltpu.VMEM((_CH, _DEGW), jnp.float32),
    ],
    compiler_params=_sc_params,
)


def _agg_body(hs, src3, dst3, zagg, aggp, sagg, src_v, dst_v, rows_v):
    cid = lax.axis_index("c")
    sid = lax.axis_index("s")
    wid = sid * _NC + cid
    zslab = pl.ds(sid * _RPS, _RPS)
    pltpu.sync_copy(zagg.at[zslab], sagg.at[zslab])
    pltpu.sync_copy(src3.at[wid], src_v)
    pltpu.sync_copy(dst3.at[wid], dst_v)
    plsc.subcore_barrier()

    def body(i, carry):
        pltpu.sync_copy(hs.at[src_v.at[i]], rows_v)
        pltpu.sync_copy(rows_v, sagg.at[dst_v.at[i]], add=True)
        return carry

    lax.fori_loop(0, _NCHUNK, body, 0)
    plsc.subcore_barrier()
    pltpu.sync_copy(sagg.at[zslab], aggp.at[cid, zslab])


_agg_call = pl.kernel(
    _agg_body,
    out_type=jax.ShapeDtypeStruct((_NC, _NP, _H), jnp.float32),
    mesh=_mesh,
    scratch_types=[
        pltpu.VMEM_SHARED((_NP, _H), jnp.float32),
        pltpu.VMEM((_NCHUNK, _CH), jnp.int32),
        pltpu.VMEM((_NCHUNK, _CH), jnp.int32),
        pltpu.VMEM((_CH, _H), jnp.float32),
    ],
    compiler_params=_sc_params,
)


def _silu(x):
    return x * jax.nn.sigmoid(x)


def _enc_body(x_ref, Wn_ref, bn_ref, gn_ref, betan_ref, degp_ref,
              h_ref, hs_ref, ns_ref, nd_ref):
    t = jnp.dot(x_ref[...], Wn_ref[...], preferred_element_type=jnp.float32)
    t = _silu(t + bn_ref[...])
    mu = jnp.mean(t, axis=-1, keepdims=True)
    var = jnp.mean((t - mu) * (t - mu), axis=-1, keepdims=True)
    h = (t - mu) * lax.rsqrt(var + 1e-5) * gn_ref[...] + betan_ref[...]
    inv = 1.0 / _DEGW
    dego = (jnp.sum(degp_ref[0, 0], axis=-1, keepdims=True)
            + jnp.sum(degp_ref[1, 0], axis=-1, keepdims=True)) * inv
    degi = (jnp.sum(degp_ref[0, 1], axis=-1, keepdims=True)
            + jnp.sum(degp_ref[1, 1], axis=-1, keepdims=True)) * inv
    ns = lax.rsqrt(jnp.maximum(dego, 1.0))
    nd = lax.rsqrt(jnp.maximum(degi, 1.0))
    h_ref[...] = h
    hs_ref[...] = h * ns
    ns_ref[...] = ns
    nd_ref[...] = nd


_enc_call = pl.pallas_call(
    _enc_body,
    out_shape=[
        jax.ShapeDtypeStruct((_NP, _H), jnp.float32),
        jax.ShapeDtypeStruct((_NP, _H), jnp.float32),
        jax.ShapeDtypeStruct((_NP, 1), jnp.float32),
        jax.ShapeDtypeStruct((_NP, 1), jnp.float32),
    ],
)


def _layer_body(aggp_ref, nd_ref, ns_ref, h_ref, W_ref, b_ref,
                ho_ref, hso_ref):
    agg = aggp_ref[0] + aggp_ref[1]
    a = agg * nd_ref[...]
    t = jnp.dot(a, W_ref[...], preferred_element_type=jnp.float32)
    h = h_ref[...] + _silu(t + b_ref[...])
    ho_ref[...] = h
    hso_ref[...] = h * ns_ref[...]


_layer_call = pl.pallas_call(
    _layer_body,
    out_shape=[
        jax.ShapeDtypeStruct((_NP, _H), jnp.float32),
        jax.ShapeDtypeStruct((_NP, _H), jnp.float32),
    ],
)


def _final_body(aggp_ref, nd_ref, h_ref, W_ref, b_ref,
                Wd1_ref, bd1_ref, Wd2_ref, bd2_ref, out_ref):
    agg = aggp_ref[0] + aggp_ref[1]
    a = agg * nd_ref[...]
    t = jnp.dot(a, W_ref[...], preferred_element_type=jnp.float32)
    h = h_ref[...] + _silu(t + b_ref[...])
    d = _silu(jnp.dot(h, Wd1_ref[...], preferred_element_type=jnp.float32)
              + bd1_ref[...])
    out_ref[...] = (jnp.dot(d, Wd2_ref[...], preferred_element_type=jnp.float32)
                    + bd2_ref[...])


_final_call = pl.pallas_call(
    _final_body,
    out_shape=jax.ShapeDtypeStruct((_NP, _OUT), jnp.float32),
)


def kernel(node_feats, edge_feats, edge_index, Wn, bn, gn, betan, We, be, ge,
           betae, convW, convb, Wd1, bd1, Wd2, bd2):
    src3 = edge_index[0].reshape(_NW, _NCHUNK, _CH)
    dst3 = edge_index[1].reshape(_NW, _NCHUNK, _CH)
    x_p = jnp.pad(node_feats, ((0, _NP - _N), (0, 0)))
    zagg = jnp.zeros((_NP, _H), jnp.float32)
    zdeg = jnp.zeros((_NP, _DEGW), jnp.float32)
    ones_hbm = jnp.ones((_CH, _DEGW), jnp.float32)

    degp = _deg_call(src3, dst3, ones_hbm, zdeg)
    h, hs, ns, nd = _enc_call(x_p, Wn, bn, gn, betan, degp)
    for i in range(_P - 1):
        aggp = _agg_call(hs, src3, dst3, zagg)
        h, hs = _layer_call(aggp, nd, ns, h, convW[i], convb[i])
    aggp = _agg_call(hs, src3, dst3, zagg)
    out_p = _final_call(aggp, nd, h, convW[_P - 1], convb[_P - 1],
                        Wd1, bd1, Wd2, bd2)
    return out_p[:_N]


# speedup vs baseline: 12.1696x; 1.4669x over previous
import functools

import jax
import jax.numpy as jnp
from jax import lax
from jax.experimental import pallas as pl
from jax.experimental.pallas import tpu as pltpu
from jax.experimental.pallas import tpu_sc as plsc

_N = 10000
_E = 320000
_DF = 128
_H = 64
_OUT = 3
_P = 4

_NP = 10240
_NC = 2
_NS = 16
_NW = _NC * _NS
_RPS = _NP // _NS

_EW = _E // _NW
_CH = 80
_NCHUNK = _EW // _CH

_DEGW = 8

_mesh = plsc.VectorSubcoreMesh(core_axis_name="c", subcore_axis_name="s")
_sc_params = pltpu.CompilerParams(use_tc_tiling_on_sc=False)


def _deg_body(src3, dst3, ones_hbm, zdeg, degp, sdego, sdegi, src_v, dst_v,
              ones_v):
    cid = lax.axis_index("c")
    sid = lax.axis_index("s")
    wid = sid * _NC + cid
    zslab = pl.ds(sid * _RPS, _RPS)
    pltpu.sync_copy(zdeg.at[zslab], sdego.at[zslab])
    pltpu.sync_copy(zdeg.at[zslab], sdegi.at[zslab])
    pltpu.sync_copy(src3.at[wid], src_v)
    pltpu.sync_copy(dst3.at[wid], dst_v)
    pltpu.sync_copy(ones_hbm, ones_v)
    plsc.subcore_barrier()

    def body(i, carry):
        pltpu.sync_copy(ones_v, sdego.at[src_v.at[i]], add=True)
        pltpu.sync_copy(ones_v, sdegi.at[dst_v.at[i]], add=True)
        return carry

    lax.fori_loop(0, _NCHUNK, body, 0)
    plsc.subcore_barrier()
    pltpu.sync_copy(sdego.at[zslab], degp.at[cid, 0, zslab])
    pltpu.sync_copy(sdegi.at[zslab], degp.at[cid, 1, zslab])


_deg_call = pl.kernel(
    _deg_body,
    out_type=jax.ShapeDtypeStruct((_NC, 2, _NP, _DEGW), jnp.float32),
    mesh=_mesh,
    scratch_types=[
        pltpu.VMEM_SHARED((_NP, _DEGW), jnp.float32),
        pltpu.VMEM_SHARED((_NP, _DEGW), jnp.float32),
        pltpu.VMEM((_NCHUNK, _CH), jnp.int32),
        pltpu.VMEM((_NCHUNK, _CH), jnp.int32),
        pltpu.VMEM((_CH, _DEGW), jnp.float32),
    ],
    compiler_params=_sc_params,
)


def _agg_body(hs, src3, dst3, zagg, aggp, sagg, src_v, dst_v, rows0, rows1,
              sem0, sem1):
    cid = lax.axis_index("c")
    sid = lax.axis_index("s")
    wid = sid * _NC + cid
    zslab = pl.ds(sid * _RPS, _RPS)
    pltpu.sync_copy(zagg.at[zslab], sagg.at[zslab])
    pltpu.sync_copy(src3.at[wid], src_v)
    pltpu.sync_copy(dst3.at[wid], dst_v)
    plsc.subcore_barrier()

    def gather(i, buf, sem):
        return pltpu.async_copy(hs.at[src_v.at[i]], buf, sem)

    def wait(i, buf, sem):
        pltpu.make_async_copy(hs.at[src_v.at[i]], buf, sem).wait()

    def scatter(i, buf):
        pltpu.sync_copy(buf, sagg.at[dst_v.at[i]], add=True)

    gather(0, rows0, sem0)
    gather(1, rows1, sem1)

    def body(j, carry):
        i = 2 * j
        wait(i, rows0, sem0)
        scatter(i, rows0)

        @pl.when(i + 2 < _NCHUNK)
        def _():
            gather(i + 2, rows0, sem0)

        wait(i + 1, rows1, sem1)
        scatter(i + 1, rows1)

        @pl.when(i + 3 < _NCHUNK)
        def _():
            gather(i + 3, rows1, sem1)

        return carry

    lax.fori_loop(0, _NCHUNK // 2, body, 0)
    if _NCHUNK % 2:
        wait(_NCHUNK - 1, rows0, sem0)
        scatter(_NCHUNK - 1, rows0)
    plsc.subcore_barrier()
    pltpu.sync_copy(sagg.at[zslab], aggp.at[cid, zslab])


_agg_call = pl.kernel(
    _agg_body,
    out_type=jax.ShapeDtypeStruct((_NC, _NP, _H), jnp.float32),
    mesh=_mesh,
    scratch_types=[
        pltpu.VMEM_SHARED((_NP, _H), jnp.float32),
        pltpu.VMEM((_NCHUNK, _CH), jnp.int32),
        pltpu.VMEM((_NCHUNK, _CH), jnp.int32),
        pltpu.VMEM((_CH, _H), jnp.float32),
        pltpu.VMEM((_CH, _H), jnp.float32),
        pltpu.SemaphoreType.DMA,
        pltpu.SemaphoreType.DMA,
    ],
    compiler_params=_sc_params,
)


def _silu(x):
    return x * jax.nn.sigmoid(x)


def _enc_body(x_ref, Wn_ref, bn_ref, gn_ref, betan_ref, degp_ref,
              h_ref, hs_ref, ns_ref, nd_ref):
    t = jnp.dot(x_ref[...], Wn_ref[...], preferred_element_type=jnp.float32)
    t = _silu(t + bn_ref[...])
    mu = jnp.mean(t, axis=-1, keepdims=True)
    var = jnp.mean((t - mu) * (t - mu), axis=-1, keepdims=True)
    h = (t - mu) * lax.rsqrt(var + 1e-5) * gn_ref[...] + betan_ref[...]
    inv = 1.0 / _DEGW
    dego = (jnp.sum(degp_ref[0, 0], axis=-1, keepdims=True)
            + jnp.sum(degp_ref[1, 0], axis=-1, keepdims=True)) * inv
    degi = (jnp.sum(degp_ref[0, 1], axis=-1, keepdims=True)
            + jnp.sum(degp_ref[1, 1], axis=-1, keepdims=True)) * inv
    ns = lax.rsqrt(jnp.maximum(dego, 1.0))
    nd = lax.rsqrt(jnp.maximum(degi, 1.0))
    h_ref[...] = h
    hs_ref[...] = h * ns
    ns_ref[...] = ns
    nd_ref[...] = nd


_enc_call = pl.pallas_call(
    _enc_body,
    out_shape=[
        jax.ShapeDtypeStruct((_NP, _H), jnp.float32),
        jax.ShapeDtypeStruct((_NP, _H), jnp.float32),
        jax.ShapeDtypeStruct((_NP, 1), jnp.float32),
        jax.ShapeDtypeStruct((_NP, 1), jnp.float32),
    ],
)


def _layer_body(aggp_ref, nd_ref, ns_ref, h_ref, W_ref, b_ref,
                ho_ref, hso_ref):
    agg = aggp_ref[0] + aggp_ref[1]
    a = agg * nd_ref[...]
    t = jnp.dot(a, W_ref[...], preferred_element_type=jnp.float32)
    h = h_ref[...] + _silu(t + b_ref[...])
    ho_ref[...] = h
    hso_ref[...] = h * ns_ref[...]


_layer_call = pl.pallas_call(
    _layer_body,
    out_shape=[
        jax.ShapeDtypeStruct((_NP, _H), jnp.float32),
        jax.ShapeDtypeStruct((_NP, _H), jnp.float32),
    ],
)


def _final_body(aggp_ref, nd_ref, h_ref, W_ref, b_ref,
                Wd1_ref, bd1_ref, Wd2_ref, bd2_ref, out_ref):
    agg = aggp_ref[0] + aggp_ref[1]
    a = agg * nd_ref[...]
    t = jnp.dot(a, W_ref[...], preferred_element_type=jnp.float32)
    h = h_ref[...] + _silu(t + b_ref[...])
    d = _silu(jnp.dot(h, Wd1_ref[...], preferred_element_type=jnp.float32)
              + bd1_ref[...])
    out_ref[...] = (jnp.dot(d, Wd2_ref[...], preferred_element_type=jnp.float32)
                    + bd2_ref[...])


_final_call = pl.pallas_call(
    _final_body,
    out_shape=jax.ShapeDtypeStruct((_NP, _OUT), jnp.float32),
)


def kernel(node_feats, edge_feats, edge_index, Wn, bn, gn, betan, We, be, ge,
           betae, convW, convb, Wd1, bd1, Wd2, bd2):
    src3 = edge_index[0].reshape(_NW, _NCHUNK, _CH)
    dst3 = edge_index[1].reshape(_NW, _NCHUNK, _CH)
    x_p = jnp.pad(node_feats, ((0, _NP - _N), (0, 0)))
    zagg = jnp.zeros((_NP, _H), jnp.float32)
    zdeg = jnp.zeros((_NP, _DEGW), jnp.float32)
    ones_hbm = jnp.ones((_CH, _DEGW), jnp.float32)

    degp = _deg_call(src3, dst3, ones_hbm, zdeg)
    h, hs, ns, nd = _enc_call(x_p, Wn, bn, gn, betan, degp)
    for i in range(_P - 1):
        aggp = _agg_call(hs, src3, dst3, zagg)
        h, hs = _layer_call(aggp, nd, ns, h, convW[i], convb[i])
    aggp = _agg_call(hs, src3, dst3, zagg)
    out_p = _final_call(aggp, nd, h, convW[_P - 1], convb[_P - 1],
                        Wd1, bd1, Wd2, bd2)
    return out_p[:_N]


# speedup vs baseline: 15.4331x; 1.2682x over previous
import functools

import jax
import jax.numpy as jnp
from jax import lax
from jax.experimental import pallas as pl
from jax.experimental.pallas import tpu as pltpu
from jax.experimental.pallas import tpu_sc as plsc

_N = 10000
_E = 320000
_DF = 128
_H = 64
_OUT = 3
_P = 4

_NP = 10240
_NC = 2
_NS = 16
_NW = _NC * _NS
_RPS = _NP // _NS

_EW = _E // _NW
_CH = 80
_NCHUNK = _EW // _CH

_DEGW = 8
_NCHUNK2 = 2 * _E // (_NW * _CH)
_DLAG = 8

_mesh = plsc.VectorSubcoreMesh(core_axis_name="c", subcore_axis_name="s")
_sc_params = pltpu.CompilerParams(use_tc_tiling_on_sc=False)


def _deg_body(sd3, ones_hbm, zdeg, degp, sdeg, sd_v, ones_v, sem):
    cid = lax.axis_index("c")
    sid = lax.axis_index("s")
    wid = sid * _NC + cid
    zslab = pl.ds(sid * 2 * _RPS, 2 * _RPS)
    pltpu.sync_copy(zdeg.at[zslab], sdeg.at[zslab])
    pltpu.sync_copy(sd3.at[wid], sd_v)
    pltpu.sync_copy(ones_hbm, ones_v)
    plsc.subcore_barrier()

    def body(i, carry):
        pltpu.async_copy(ones_v, sdeg.at[sd_v.at[i]], sem, add=True)

        @pl.when(i >= _DLAG)
        def _():
            pltpu.make_async_copy(ones_v, sdeg.at[sd_v.at[i - _DLAG]],
                                  sem).wait()

        return carry

    lax.fori_loop(0, _NCHUNK2, body, 0)
    for i in range(_NCHUNK2 - _DLAG, _NCHUNK2):
        pltpu.make_async_copy(ones_v, sdeg.at[sd_v.at[i]], sem).wait()
    plsc.subcore_barrier()
    pltpu.sync_copy(sdeg.at[zslab], degp.at[cid, zslab])


_deg_call = pl.kernel(
    _deg_body,
    out_type=jax.ShapeDtypeStruct((_NC, 2 * _NP, _DEGW), jnp.float32),
    mesh=_mesh,
    scratch_types=[
        pltpu.VMEM_SHARED((2 * _NP, _DEGW), jnp.float32),
        pltpu.VMEM((_NCHUNK2, _CH), jnp.int32),
        pltpu.VMEM((_CH, _DEGW), jnp.float32),
        pltpu.SemaphoreType.DMA,
    ],
    compiler_params=_sc_params,
)


_NBUF = 4


def _agg_body(hs, src3, dst3, zagg, aggp, sagg, src_v, dst_v, *rest):
    rows = rest[:_NBUF]
    sems = rest[_NBUF:]
    cid = lax.axis_index("c")
    sid = lax.axis_index("s")
    wid = sid * _NC + cid
    zslab = pl.ds(sid * _RPS, _RPS)
    pltpu.sync_copy(zagg.at[zslab], sagg.at[zslab])
    pltpu.sync_copy(src3.at[wid], src_v)
    pltpu.sync_copy(dst3.at[wid], dst_v)
    plsc.subcore_barrier()

    def gather(i, buf, sem):
        return pltpu.async_copy(hs.at[src_v.at[i]], buf, sem)

    def wait(i, buf, sem):
        pltpu.make_async_copy(hs.at[src_v.at[i]], buf, sem).wait()

    def scatter(i, buf):
        pltpu.sync_copy(buf, sagg.at[dst_v.at[i]], add=True)

    bufs = list(zip(rows, sems))
    for b in range(_NBUF):
        gather(b, *bufs[b])

    def body(j, carry):
        for b in range(_NBUF):
            i = _NBUF * j + b
            wait(i, *bufs[b])
            scatter(i, bufs[b][0])

            @pl.when(i + _NBUF < _NCHUNK)
            def _():
                gather(i + _NBUF, *bufs[b])

        return carry

    lax.fori_loop(0, _NCHUNK // _NBUF, body, 0)
    for i in range((_NCHUNK // _NBUF) * _NBUF, _NCHUNK):
        wait(i, *bufs[i % _NBUF])
        scatter(i, bufs[i % _NBUF][0])
    plsc.subcore_barrier()
    pltpu.sync_copy(sagg.at[zslab], aggp.at[cid, zslab])


_agg_call = pl.kernel(
    _agg_body,
    out_type=jax.ShapeDtypeStruct((_NC, _NP, _H), jnp.float32),
    mesh=_mesh,
    scratch_types=[
        pltpu.VMEM_SHARED((_NP, _H), jnp.float32),
        pltpu.VMEM((_NCHUNK, _CH), jnp.int32),
        pltpu.VMEM((_NCHUNK, _CH), jnp.int32),
    ] + [pltpu.VMEM((_CH, _H), jnp.float32)] * _NBUF
      + [pltpu.SemaphoreType.DMA] * _NBUF,
    compiler_params=_sc_params,
)


def _silu(x):
    return x * jax.nn.sigmoid(x)


def _enc_body(x_ref, Wn_ref, bn_ref, gn_ref, betan_ref, degp_ref,
              h_ref, hs_ref, ns_ref, nd_ref):
    t = jnp.dot(x_ref[...], Wn_ref[...], preferred_element_type=jnp.float32)
    t = _silu(t + bn_ref[...])
    mu = jnp.mean(t, axis=-1, keepdims=True)
    var = jnp.mean((t - mu) * (t - mu), axis=-1, keepdims=True)
    h = (t - mu) * lax.rsqrt(var + 1e-5) * gn_ref[...] + betan_ref[...]
    inv = 1.0 / _DEGW
    dego = (jnp.sum(degp_ref[0, :_NP], axis=-1, keepdims=True)
            + jnp.sum(degp_ref[1, :_NP], axis=-1, keepdims=True)) * inv
    degi = (jnp.sum(degp_ref[0, _NP:], axis=-1, keepdims=True)
            + jnp.sum(degp_ref[1, _NP:], axis=-1, keepdims=True)) * inv
    ns = lax.rsqrt(jnp.maximum(dego, 1.0))
    nd = lax.rsqrt(jnp.maximum(degi, 1.0))
    h_ref[...] = h
    hs_ref[...] = h * ns
    ns_ref[...] = ns
    nd_ref[...] = nd


_enc_call = pl.pallas_call(
    _enc_body,
    out_shape=[
        jax.ShapeDtypeStruct((_NP, _H), jnp.float32),
        jax.ShapeDtypeStruct((_NP, _H), jnp.float32),
        jax.ShapeDtypeStruct((_NP, 1), jnp.float32),
        jax.ShapeDtypeStruct((_NP, 1), jnp.float32),
    ],
)


def _layer_body(aggp_ref, nd_ref, ns_ref, h_ref, W_ref, b_ref,
                ho_ref, hso_ref):
    agg = aggp_ref[0] + aggp_ref[1]
    a = agg * nd_ref[...]
    t = jnp.dot(a, W_ref[...], preferred_element_type=jnp.float32)
    h = h_ref[...] + _silu(t + b_ref[...])
    ho_ref[...] = h
    hso_ref[...] = h * ns_ref[...]


_layer_call = pl.pallas_call(
    _layer_body,
    out_shape=[
        jax.ShapeDtypeStruct((_NP, _H), jnp.float32),
        jax.ShapeDtypeStruct((_NP, _H), jnp.float32),
    ],
)


def _final_body(aggp_ref, nd_ref, h_ref, W_ref, b_ref,
                Wd1_ref, bd1_ref, Wd2_ref, bd2_ref, out_ref):
    agg = aggp_ref[0] + aggp_ref[1]
    a = agg * nd_ref[...]
    t = jnp.dot(a, W_ref[...], preferred_element_type=jnp.float32)
    h = h_ref[...] + _silu(t + b_ref[...])
    d = _silu(jnp.dot(h, Wd1_ref[...], preferred_element_type=jnp.float32)
              + bd1_ref[...])
    out_ref[...] = (jnp.dot(d, Wd2_ref[...], preferred_element_type=jnp.float32)
                    + bd2_ref[...])


_final_call = pl.pallas_call(
    _final_body,
    out_shape=jax.ShapeDtypeStruct((_NP, _OUT), jnp.float32),
)


def kernel(node_feats, edge_feats, edge_index, Wn, bn, gn, betan, We, be, ge,
           betae, convW, convb, Wd1, bd1, Wd2, bd2):
    src3 = edge_index[0].reshape(_NW, _NCHUNK, _CH)
    dst3 = edge_index[1].reshape(_NW, _NCHUNK, _CH)
    sd3 = jnp.concatenate([edge_index[0], edge_index[1] + _NP]
                          ).reshape(_NW, _NCHUNK2, _CH)
    x_p = jnp.pad(node_feats, ((0, _NP - _N), (0, 0)))
    zagg = jnp.zeros((_NP, _H), jnp.float32)
    zdeg = jnp.zeros((2 * _NP, _DEGW), jnp.float32)
    ones_hbm = jnp.ones((_CH, _DEGW), jnp.float32)

    degp = _deg_call(sd3, ones_hbm, zdeg)
    h, hs, ns, nd = _enc_call(x_p, Wn, bn, gn, betan, degp)
    for i in range(_P - 1):
        aggp = _agg_call(hs, src3, dst3, zagg)
        h, hs = _layer_call(aggp, nd, ns, h, convW[i], convb[i])
    aggp = _agg_call(hs, src3, dst3, zagg)
    out_p = _final_call(aggp, nd, h, convW[_P - 1], convb[_P - 1],
                        Wd1, bd1, Wd2, bd2)
    return out_p[:_N]
